# baseline (speedup 1.0000x reference)
import functools

import jax
import jax.numpy as jnp
from jax import lax
from jax.experimental import pallas as pl
from jax.experimental.pallas import tpu as pltpu, tpu_sc as plsc

_NUM_CORES = 2


@functools.cache
def _sc_copy(n):
    mesh = plsc.ScalarSubcoreMesh(axis_name="c", num_cores=1)

    @functools.partial(
        pl.kernel,
        mesh=mesh,
        out_type=(
            jax.ShapeDtypeStruct((n,), jnp.float32),
            jax.ShapeDtypeStruct((n,), jnp.float32),
        ),
        scratch_types=[
            pltpu.VMEM_SHARED((n,), jnp.float32),
            pltpu.VMEM_SHARED((n,), jnp.float32),
            pltpu.SemaphoreType.DMA,
            pltpu.SemaphoreType.DMA,
            pltpu.SemaphoreType.DMA,
            pltpu.SemaphoreType.DMA,
        ],
    )
    def body(kx_hbm, vx_hbm, k_out_hbm, v_out_hbm, kbuf, vbuf, s0, s1, s2, s3):
        half = n // 2
        lo = pl.ds(0, half)
        hi = pl.ds(half, half)
        loads = (
            pltpu.make_async_copy(kx_hbm.at[lo], kbuf.at[lo], s0),
            pltpu.make_async_copy(vx_hbm.at[lo], vbuf.at[lo], s1),
            pltpu.make_async_copy(kx_hbm.at[hi], kbuf.at[hi], s2),
            pltpu.make_async_copy(vx_hbm.at[hi], vbuf.at[hi], s3),
        )
        stores = (
            pltpu.make_async_copy(kbuf.at[lo], k_out_hbm.at[lo], s0),
            pltpu.make_async_copy(vbuf.at[lo], v_out_hbm.at[lo], s1),
            pltpu.make_async_copy(kbuf.at[hi], k_out_hbm.at[hi], s2),
            pltpu.make_async_copy(vbuf.at[hi], v_out_hbm.at[hi], s3),
        )
        for ld in loads:
            ld.start()
        for ld, st in zip(loads, stores):
            ld.wait()
            st.start()
        for st in stores:
            st.wait()

    return body


def kernel(kx, vx, k_cache, v_cache):
    del k_cache, v_cache
    shape = kx.shape
    n = kx.size
    k_flat, v_flat = _sc_copy(n)(kx.reshape(n), vx.reshape(n))
    return k_flat.reshape(shape), v_flat.reshape(shape)


# speedup vs baseline: 1.0016x; 1.0016x over previous
import functools

import jax
import jax.numpy as jnp
from jax import lax
from jax.experimental import pallas as pl
from jax.experimental.pallas import tpu as pltpu, tpu_sc as plsc

_NUM_CORES = 2


@functools.cache
def _sc_copy(n):
    mesh = plsc.ScalarSubcoreMesh(axis_name="c", num_cores=1)

    @functools.partial(
        pl.kernel,
        mesh=mesh,
        out_type=(
            jax.ShapeDtypeStruct((n,), jnp.float32),
            jax.ShapeDtypeStruct((n,), jnp.float32),
        ),
        scratch_types=[
            pltpu.VMEM_SHARED((n,), jnp.float32),
            pltpu.VMEM_SHARED((n,), jnp.float32),
            pltpu.SemaphoreType.DMA,
            pltpu.SemaphoreType.DMA,
        ],
    )
    def body(kx_hbm, vx_hbm, k_out_hbm, v_out_hbm, kbuf, vbuf, ksem, vsem):
        kld = pltpu.make_async_copy(kx_hbm, kbuf, ksem)
        vld = pltpu.make_async_copy(vx_hbm, vbuf, vsem)
        kld.start()
        vld.start()
        kld.wait()
        kst = pltpu.make_async_copy(kbuf, k_out_hbm, ksem)
        kst.start()
        vld.wait()
        vst = pltpu.make_async_copy(vbuf, v_out_hbm, vsem)
        vst.start()
        kst.wait()
        vst.wait()

    return body


def kernel(kx, vx, k_cache, v_cache):
    del k_cache, v_cache
    shape = kx.shape
    n = kx.size
    k_flat, v_flat = _sc_copy(n)(kx.reshape(n), vx.reshape(n))
    return k_flat.reshape(shape), v_flat.reshape(shape)
